# baseline (speedup 1.0000x reference)
import functools

import jax
import jax.numpy as jnp
from jax import lax
from jax.experimental import pallas as pl
from jax.experimental.pallas import tpu as pltpu
from jax.experimental.pallas import tpu_sc as plsc

N = 100000
D = 128
B = 256
NCLS = 7
K = B * NCLS
CH = 80
NCHUNK = N // CH
NW = 32
KBASE = NCHUNK // NW
KEXTRA = NCHUNK - KBASE * NW
KMAX = KBASE + 1
NB = 6
ROWS_PT = K // 16
HCH = 2500
NHCH = N // HCH

_mesh = plsc.VectorSubcoreMesh(core_axis_name="c", subcore_axis_name="s")


@functools.partial(
    pl.kernel,
    mesh=_mesh,
    out_type=jax.ShapeDtypeStruct((2, K, D), jnp.float32),
    scratch_types=[
        pltpu.VMEM((KMAX, CH), jnp.int32),
        pltpu.VMEM((KMAX * CH,), jnp.int32),
        pltpu.VMEM((KMAX * CH,), jnp.int32),
        pltpu.VMEM((NB, CH, D), jnp.float32),
        pltpu.VMEM((16, D), jnp.float32),
        pltpu.VMEM_SHARED((K, D), jnp.float32),
        pltpu.SemaphoreType.DMA,
        pltpu.SemaphoreType.DMA,
        pltpu.SemaphoreType.DMA,
        pltpu.SemaphoreType.DMA,
        pltpu.SemaphoreType.DMA,
        pltpu.SemaphoreType.DMA,
    ],
)
def _sc_scatter(h_hbm, seg_hbm, pos_hbm, tab_out,
                keys_v, seg_v, pos_v, ring_v, zero_v, table_sh,
                sem0, sem1, sem2, sem3, sem4, sem5):
    cid = lax.axis_index("c")
    sid = lax.axis_index("s")
    wid = sid * 2 + cid
    start = wid * KBASE + jnp.minimum(wid, KEXTRA)
    cnt = KBASE + (wid < KEXTRA).astype(jnp.int32)
    row0 = start * CH
    sems = (sem0, sem1, sem2, sem3, sem4, sem5)

    def load(kk, b):
        pltpu.make_async_copy(
            h_hbm.at[pl.ds((start + kk) * CH, CH)], ring_v.at[b], sems[b]
        ).start()

    for b in range(NB):
        @pl.when(b < cnt)
        def _(b=b):
            load(b, b)

    pltpu.sync_copy(seg_hbm.at[pl.ds(row0, KBASE * CH)],
                    seg_v.at[pl.ds(0, KBASE * CH)])
    pltpu.sync_copy(pos_hbm.at[pl.ds(row0, KBASE * CH)],
                    pos_v.at[pl.ds(0, KBASE * CH)])

    @pl.when(wid < KEXTRA)
    def _():
        pltpu.sync_copy(seg_hbm.at[pl.ds(row0 + KBASE * CH, CH)],
                        seg_v.at[pl.ds(KBASE * CH, CH)])
        pltpu.sync_copy(pos_hbm.at[pl.ds(row0 + KBASE * CH, CH)],
                        pos_v.at[pl.ds(KBASE * CH, CH)])

    nv = CH // 16

    def fill_keys(i, _):
        sl = pl.ds(i * 16, 16)
        keys_v[i // nv, pl.ds((i % nv) * 16, 16)] = (
            seg_v[sl] * NCLS + pos_v[sl])
        return 0
    lax.fori_loop(0, KBASE * CH // 16, fill_keys, 0)

    @pl.when(wid < KEXTRA)
    def _():
        def fill_tail(i, _):
            sl = pl.ds(KBASE * CH + i * 16, 16)
            keys_v[KBASE, pl.ds(i * 16, 16)] = seg_v[sl] * NCLS + pos_v[sl]
            return 0
        lax.fori_loop(0, nv, fill_tail, 0)

    z16 = jnp.zeros((16,), jnp.float32)

    def fill_zero(i, _):
        zero_v[i // 8, pl.ds((i % 8) * 16, 16)] = z16
        return 0
    lax.fori_loop(0, 16 * (D // 16), fill_zero, 0)

    for j in range(ROWS_PT // 16):
        pltpu.sync_copy(zero_v, table_sh.at[pl.ds(sid * ROWS_PT + j * 16, 16)])
    plsc.subcore_barrier()

    def wait_load(b):
        pltpu.make_async_copy(
            h_hbm.at[pl.ds(0, CH)], ring_v.at[b], sems[b]
        ).wait()

    def outer(o, _):
        for b in range(NB):
            kk = o * NB + b

            @pl.when(kk < cnt)
            def _(b=b, kk=kk):
                wait_load(b)
                pltpu.sync_copy(ring_v.at[b], table_sh.at[keys_v.at[kk]],
                                add=True)

                @pl.when(kk + NB < cnt)
                def _(b=b, kk=kk):
                    load(kk + NB, b)
        return 0
    lax.fori_loop(0, (KMAX + NB - 1) // NB, outer, 0)
    plsc.subcore_barrier()

    pltpu.sync_copy(table_sh.at[pl.ds(sid * ROWS_PT, ROWS_PT)],
                    tab_out.at[cid, pl.ds(sid * ROWS_PT, ROWS_PT)])


def _hist_body(seg_ref, cnt_ref):
    i = pl.program_id(0)

    @pl.when(i == 0)
    def _():
        cnt_ref[:, :] = jnp.zeros((B, 128), jnp.float32)

    row = seg_ref[0]
    ids = lax.broadcasted_iota(jnp.int32, (B, HCH), 0)
    eq = (row == ids).astype(jnp.float32)
    s = jnp.sum(eq, axis=1, keepdims=True)
    cnt_ref[:, :] += jnp.broadcast_to(s, (B, 128))


def _finalize_body(t0_ref, t1_ref, cnt_ref, o_ref):
    norm = jnp.maximum(cnt_ref[:, 0:1], 1.0)
    o_ref[:, :] = (t0_ref[:, :] + t1_ref[:, :]) / norm


@jax.jit
def _run(h, seg32, pos32):
    tab = _sc_scatter(h, seg32, pos32)

    seg3d = seg32.reshape(NHCH, 1, HCH)
    counts = pl.pallas_call(
        _hist_body,
        grid=(NHCH,),
        in_specs=[pl.BlockSpec((1, 1, HCH), lambda i: (i, 0, 0))],
        out_specs=pl.BlockSpec((B, 128), lambda i: (0, 0)),
        out_shape=jax.ShapeDtypeStruct((B, 128), jnp.float32),
    )(seg3d)

    t0 = tab[0].reshape(B, NCLS * D)
    t1 = tab[1].reshape(B, NCLS * D)
    return pl.pallas_call(
        _finalize_body,
        out_shape=jax.ShapeDtypeStruct((B, NCLS * D), jnp.float32),
    )(t0, t1, counts)


def kernel(h, pos, segment_ids, num_segments):
    del num_segments
    return _run(h, segment_ids.astype(jnp.int32), pos.astype(jnp.int32))


# speedup vs baseline: 1.0088x; 1.0088x over previous
import functools

import jax
import jax.numpy as jnp
from jax import lax
from jax.experimental import pallas as pl
from jax.experimental.pallas import tpu as pltpu
from jax.experimental.pallas import tpu_sc as plsc

N = 100000
D = 128
B = 256
NCLS = 7
K = B * NCLS
CH = 80
NCHUNK = N // CH
NW = 32
KBASE = NCHUNK // NW
KEXTRA = NCHUNK - KBASE * NW
KMAX = KBASE + 1
NB = 4
ROWS_PT = K // 16
HCH = 2500
NHCH = N // HCH

_mesh = plsc.VectorSubcoreMesh(core_axis_name="c", subcore_axis_name="s")


@functools.partial(
    pl.kernel,
    mesh=_mesh,
    out_type=jax.ShapeDtypeStruct((2, K, D), jnp.float32),
    scratch_types=[
        pltpu.VMEM((KMAX, CH), jnp.int32),
        pltpu.VMEM((KMAX * CH,), jnp.int32),
        pltpu.VMEM((KMAX * CH,), jnp.int32),
        pltpu.VMEM((NB, CH, D), jnp.float32),
        pltpu.VMEM((16, D), jnp.float32),
        pltpu.VMEM_SHARED((K, D), jnp.float32),
        pltpu.SemaphoreType.DMA,
        pltpu.SemaphoreType.DMA,
        pltpu.SemaphoreType.DMA,
        pltpu.SemaphoreType.DMA,
    ],
)
def _sc_scatter(h_hbm, seg_hbm, pos_hbm, tab_out,
                keys_v, seg_v, pos_v, ring_v, zero_v, table_sh,
                sem0, sem1, sem2, sem3):
    cid = lax.axis_index("c")
    sid = lax.axis_index("s")
    wid = sid * 2 + cid
    start = wid * KBASE + jnp.minimum(wid, KEXTRA)
    cnt = KBASE + (wid < KEXTRA).astype(jnp.int32)
    row0 = start * CH
    sems = (sem0, sem1, sem2, sem3)

    def load(kk, b):
        pltpu.make_async_copy(
            h_hbm.at[pl.ds((start + kk) * CH, CH)], ring_v.at[b], sems[b]
        ).start()

    for b in range(NB):
        @pl.when(b < cnt)
        def _(b=b):
            load(b, b)

    pltpu.sync_copy(seg_hbm.at[pl.ds(row0, KBASE * CH)],
                    seg_v.at[pl.ds(0, KBASE * CH)])
    pltpu.sync_copy(pos_hbm.at[pl.ds(row0, KBASE * CH)],
                    pos_v.at[pl.ds(0, KBASE * CH)])

    @pl.when(wid < KEXTRA)
    def _():
        pltpu.sync_copy(seg_hbm.at[pl.ds(row0 + KBASE * CH, CH)],
                        seg_v.at[pl.ds(KBASE * CH, CH)])
        pltpu.sync_copy(pos_hbm.at[pl.ds(row0 + KBASE * CH, CH)],
                        pos_v.at[pl.ds(KBASE * CH, CH)])

    nv = CH // 16

    def fill_keys(i, _):
        sl = pl.ds(i * 16, 16)
        keys_v[i // nv, pl.ds((i % nv) * 16, 16)] = (
            seg_v[sl] * NCLS + pos_v[sl])
        return 0
    lax.fori_loop(0, KBASE * CH // 16, fill_keys, 0)

    @pl.when(wid < KEXTRA)
    def _():
        def fill_tail(i, _):
            sl = pl.ds(KBASE * CH + i * 16, 16)
            keys_v[KBASE, pl.ds(i * 16, 16)] = seg_v[sl] * NCLS + pos_v[sl]
            return 0
        lax.fori_loop(0, nv, fill_tail, 0)

    z16 = jnp.zeros((16,), jnp.float32)

    def fill_zero(i, _):
        zero_v[i // 8, pl.ds((i % 8) * 16, 16)] = z16
        return 0
    lax.fori_loop(0, 16 * (D // 16), fill_zero, 0)

    for j in range(ROWS_PT // 16):
        pltpu.sync_copy(zero_v, table_sh.at[pl.ds(sid * ROWS_PT + j * 16, 16)])
    plsc.subcore_barrier()

    def wait_load(b):
        pltpu.make_async_copy(
            h_hbm.at[pl.ds(0, CH)], ring_v.at[b], sems[b]
        ).wait()

    def outer(o, _):
        for b in range(NB):
            kk = o * NB + b

            @pl.when(kk < cnt)
            def _(b=b, kk=kk):
                wait_load(b)
                pltpu.sync_copy(ring_v.at[b], table_sh.at[keys_v.at[kk]],
                                add=True)

                @pl.when(kk + NB < cnt)
                def _(b=b, kk=kk):
                    load(kk + NB, b)
        return 0
    lax.fori_loop(0, KMAX // NB, outer, 0)
    plsc.subcore_barrier()

    pltpu.sync_copy(table_sh.at[pl.ds(sid * ROWS_PT, ROWS_PT)],
                    tab_out.at[cid, pl.ds(sid * ROWS_PT, ROWS_PT)])


def _hist_body(seg_ref, cnt_ref):
    i = pl.program_id(0)

    @pl.when(i == 0)
    def _():
        cnt_ref[:, :] = jnp.zeros((B, 128), jnp.float32)

    row = seg_ref[0]
    ids = lax.broadcasted_iota(jnp.int32, (B, HCH), 0)
    eq = (row == ids).astype(jnp.float32)
    s = jnp.sum(eq, axis=1, keepdims=True)
    cnt_ref[:, :] += jnp.broadcast_to(s, (B, 128))


def _finalize_body(t0_ref, t1_ref, cnt_ref, o_ref):
    norm = jnp.maximum(cnt_ref[:, 0:1], 1.0)
    o_ref[:, :] = (t0_ref[:, :] + t1_ref[:, :]) / norm


@jax.jit
def _run(h, seg32, pos32):
    tab = _sc_scatter(h, seg32, pos32)

    seg3d = seg32.reshape(NHCH, 1, HCH)
    counts = pl.pallas_call(
        _hist_body,
        grid=(NHCH,),
        in_specs=[pl.BlockSpec((1, 1, HCH), lambda i: (i, 0, 0))],
        out_specs=pl.BlockSpec((B, 128), lambda i: (0, 0)),
        out_shape=jax.ShapeDtypeStruct((B, 128), jnp.float32),
    )(seg3d)

    t0 = tab[0].reshape(B, NCLS * D)
    t1 = tab[1].reshape(B, NCLS * D)
    return pl.pallas_call(
        _finalize_body,
        out_shape=jax.ShapeDtypeStruct((B, NCLS * D), jnp.float32),
    )(t0, t1, counts)


def kernel(h, pos, segment_ids, num_segments):
    del num_segments
    return _run(h, segment_ids.astype(jnp.int32), pos.astype(jnp.int32))
